# baseline (speedup 1.0000x reference)
import functools

import jax
import jax.numpy as jnp
from jax import lax
from jax.experimental import pallas as pl
from jax.experimental.pallas import tpu as pltpu
from jax.experimental.pallas import tpu_sc as plsc

N_ROWS = 1000001
EMBED_DIM = 64
BATCH = 16384

_info = plsc.get_sparse_core_info()
_NC, _NS, _L = _info.num_cores, _info.num_subcores, _info.num_lanes
_NW = _NC * _NS
_B_PER_W = BATCH // _NW
_CHUNK = _L
_N_CHUNKS = _B_PER_W // _CHUNK


def _gather_body(idx_hbm, tab_hbm, out_hbm, idx_v, tile_v, out_v, *sems):
    wid = lax.axis_index("s") * _NC + lax.axis_index("c")
    base = wid * _B_PER_W
    pltpu.sync_copy(idx_hbm.at[pl.ds(base, _B_PER_W)], idx_v)

    def _fire(ch, b):
        v = idx_v[pl.ds(ch * _CHUNK, _CHUNK)]
        tv = lax.shift_right_logical(v, 3)
        ts = [tv[k] for k in range(_CHUNK)]
        for k in range(_CHUNK):
            pltpu.async_copy(
                tab_hbm.at[pl.ds(ts[k], 1), :, :],
                tile_v.at[pl.ds(b * _CHUNK + k, 1), :, :],
                sems[4 * b + (k % 4)],
            )

    def _wait(b):
        for h in range(4):
            pltpu.make_async_copy(
                tab_hbm.at[pl.ds(0, _CHUNK // 4), :, :],
                tile_v.at[pl.ds(b * _CHUNK, _CHUNK // 4), :, :],
                sems[4 * b + h],
            ).wait()

    def _extract(ch, b):
        v = idx_v[pl.ds(ch * _CHUNK, _CHUNK)]
        uv = lax.bitwise_and(v, 7)
        us = [uv[k] for k in range(_CHUNK)]
        for k in range(_CHUNK):
            dst = ch * _CHUNK + k
            for c in range(0, EMBED_DIM, _L):
                out_v[dst, pl.ds(c, _L)] = tile_v[
                    b * _CHUNK + k, us[k], pl.ds(c, _L)
                ]

    _fire(0, 0)
    _fire(1, 1)

    def _step(s, carry):
        ch0 = s * 2
        _wait(0)
        _extract(ch0, 0)
        _fire(ch0 + 2, 0)
        _wait(1)
        _extract(ch0 + 1, 1)
        _fire(ch0 + 3, 1)
        return carry

    lax.fori_loop(0, _N_CHUNKS // 2 - 1, _step, 0)
    _wait(0)
    _extract(_N_CHUNKS - 2, 0)
    _wait(1)
    _extract(_N_CHUNKS - 1, 1)

    pltpu.sync_copy(out_v, out_hbm.at[pl.ds(base, _B_PER_W), :])


_mesh = plsc.VectorSubcoreMesh(core_axis_name="c", subcore_axis_name="s")

_gather = functools.partial(
    pl.kernel,
    mesh=_mesh,
    out_type=jax.ShapeDtypeStruct((BATCH, EMBED_DIM), jnp.float32),
    scratch_types=[
        pltpu.VMEM((_B_PER_W,), jnp.int32),
        pltpu.VMEM((2 * _CHUNK, 8, EMBED_DIM), jnp.float32),
        pltpu.VMEM((_B_PER_W, EMBED_DIM), jnp.float32),
        pltpu.SemaphoreType.DMA,
        pltpu.SemaphoreType.DMA,
        pltpu.SemaphoreType.DMA,
        pltpu.SemaphoreType.DMA,
        pltpu.SemaphoreType.DMA,
        pltpu.SemaphoreType.DMA,
        pltpu.SemaphoreType.DMA,
        pltpu.SemaphoreType.DMA,
    ],
    compiler_params=pltpu.CompilerParams(needs_layout_passes=False),
)(_gather_body)


def kernel(nodes, table):
    tab3 = table[: N_ROWS - 1].reshape((N_ROWS - 1) // 8, 8, EMBED_DIM)
    return _gather(nodes.astype(jnp.int32), tab3)


# speedup vs baseline: 1.0037x; 1.0037x over previous
import functools

import jax
import jax.numpy as jnp
from jax import lax
from jax.experimental import pallas as pl
from jax.experimental.pallas import tpu as pltpu
from jax.experimental.pallas import tpu_sc as plsc

N_ROWS = 1000001
EMBED_DIM = 64
BATCH = 16384

_info = plsc.get_sparse_core_info()
_NC, _NS, _L = _info.num_cores, _info.num_subcores, _info.num_lanes
_NW = _NC * _NS
_B_PER_W = BATCH // _NW
_CHUNK = _L
_N_CHUNKS = _B_PER_W // _CHUNK
_NBUF = 3


def _gather_body(idx_hbm, tab_hbm, out_hbm, idx_v, tile_v, out_v, *sems):
    wid = lax.axis_index("s") * _NC + lax.axis_index("c")
    base = wid * _B_PER_W
    pltpu.sync_copy(idx_hbm.at[pl.ds(base, _B_PER_W)], idx_v)

    def _fire(ch, b):
        v = idx_v[pl.ds(ch * _CHUNK, _CHUNK)]
        tv = lax.shift_right_logical(v, 3)
        ts = [tv[k] for k in range(_CHUNK)]
        for k in range(_CHUNK):
            pltpu.async_copy(
                tab_hbm.at[pl.ds(ts[k], 1), :, :],
                tile_v.at[pl.ds(b * _CHUNK + k, 1), :, :],
                sems[b],
            )

    def _wait(b):
        pltpu.make_async_copy(
            tab_hbm.at[pl.ds(0, _CHUNK), :, :],
            tile_v.at[pl.ds(b * _CHUNK, _CHUNK), :, :],
            sems[b],
        ).wait()

    def _extract(ch, b):
        v = idx_v[pl.ds(ch * _CHUNK, _CHUNK)]
        uv = lax.bitwise_and(v, 7)
        us = [uv[k] for k in range(_CHUNK)]
        for k in range(_CHUNK):
            dst = ch * _CHUNK + k
            for c in range(0, EMBED_DIM, _L):
                out_v[dst, pl.ds(c, _L)] = tile_v[
                    b * _CHUNK + k, us[k], pl.ds(c, _L)
                ]

    _fire(0, 0)
    _fire(1, 1)
    _fire(2, 2)

    def _step(s, carry):
        for q in range(_NBUF):
            ch = _NBUF * s + q
            _wait(q)
            _extract(ch - _NBUF, q)
            _fire(ch, q)
        return carry

    lax.fori_loop(1, (_N_CHUNKS - 2) // _NBUF, _step, 0)
    _wait(0)
    _extract(_N_CHUNKS - 5, 0)
    _fire(_N_CHUNKS - 2, 0)
    _wait(1)
    _extract(_N_CHUNKS - 4, 1)
    _fire(_N_CHUNKS - 1, 1)
    _wait(2)
    _extract(_N_CHUNKS - 3, 2)
    _wait(0)
    _extract(_N_CHUNKS - 2, 0)
    _wait(1)
    _extract(_N_CHUNKS - 1, 1)

    pltpu.sync_copy(out_v, out_hbm.at[pl.ds(base, _B_PER_W), :])


_mesh = plsc.VectorSubcoreMesh(core_axis_name="c", subcore_axis_name="s")

_gather = functools.partial(
    pl.kernel,
    mesh=_mesh,
    out_type=jax.ShapeDtypeStruct((BATCH, EMBED_DIM), jnp.float32),
    scratch_types=[
        pltpu.VMEM((_B_PER_W,), jnp.int32),
        pltpu.VMEM((_NBUF * _CHUNK, 8, EMBED_DIM), jnp.float32),
        pltpu.VMEM((_B_PER_W, EMBED_DIM), jnp.float32),
        pltpu.SemaphoreType.DMA,
        pltpu.SemaphoreType.DMA,
        pltpu.SemaphoreType.DMA,
    ],
    compiler_params=pltpu.CompilerParams(needs_layout_passes=False),
)(_gather_body)


def kernel(nodes, table):
    tab3 = table[: N_ROWS - 1].reshape((N_ROWS - 1) // 8, 8, EMBED_DIM)
    return _gather(nodes.astype(jnp.int32), tab3)
